# baseline (speedup 1.0000x reference)
import functools

import jax
import jax.numpy as jnp
from jax.experimental import pallas as pl
from jax.experimental.pallas import tpu as pltpu
from jax.experimental.pallas import tpu_sc as plsc

_NUM_TREE_NODES = 262143
_LEAF_START = 131071
_RANK = 32
_TBLK = 1024
_GWIN = 128


def _transform_body(t_ref, w1_ref, b1_ref, w2_ref, b2_ref, o0_ref, o1_ref):
    i = pl.program_id(0)
    x = t_ref[...]
    xb = x.astype(jnp.bfloat16)
    h = jax.lax.dot_general(xb, w1_ref[...], (((1,), (0,)), ((), ())),
                            preferred_element_type=jnp.float32)
    h = jnp.maximum(h + b1_ref[...], 0.0).astype(jnp.bfloat16)
    h2 = jax.lax.dot_general(h, w2_ref[...], (((1,), (0,)), ((), ())),
                             preferred_element_type=jnp.float32)
    h2 = h2 + b2_ref[...]
    rows = i * _TBLK + jax.lax.broadcasted_iota(jnp.int32, (_TBLK, 1), 0)
    out = jnp.where(rows < _LEAF_START, h2, x)
    o0_ref[...] = out[:, :_RANK]
    o1_ref[...] = out[:, _RANK:]


def _transform_table(table, w1bd, b1cat, w2bd, b2cat):
    n = table.shape[0]
    grid = (pl.cdiv(n, _TBLK),)
    return pl.pallas_call(
        _transform_body,
        grid=grid,
        in_specs=[
            pl.BlockSpec((_TBLK, 2 * _RANK), lambda i: (i, 0)),
            pl.BlockSpec((2 * _RANK, 2 * _RANK), lambda i: (0, 0)),
            pl.BlockSpec((1, 2 * _RANK), lambda i: (0, 0)),
            pl.BlockSpec((2 * _RANK, 2 * _RANK), lambda i: (0, 0)),
            pl.BlockSpec((1, 2 * _RANK), lambda i: (0, 0)),
        ],
        out_specs=[
            pl.BlockSpec((_TBLK, _RANK), lambda i: (i, 0)),
            pl.BlockSpec((_TBLK, _RANK), lambda i: (i, 0)),
        ],
        out_shape=[
            jax.ShapeDtypeStruct((n, _RANK), jnp.float32),
            jax.ShapeDtypeStruct((n, _RANK), jnp.float32),
        ],
    )(table, w1bd, b1cat, w2bd, b2cat)


_NW = 32
_KROWS = 8
_CHUNK = _KROWS * 128


def _sc_gather2(t0, t1, idx_flat):
    m = idx_flat.shape[0]
    b_per_w = m // _NW
    n_chunks = b_per_w // _CHUNK
    mesh = plsc.VectorSubcoreMesh(core_axis_name="c", subcore_axis_name="s")

    @functools.partial(
        pl.kernel,
        mesh=mesh,
        out_type=(
            jax.ShapeDtypeStruct((m, _RANK), jnp.float32),
            jax.ShapeDtypeStruct((m, _RANK), jnp.float32),
        ),
        scratch_types=[
            pltpu.VMEM((_KROWS, 128), jnp.int32),
            pltpu.VMEM((_CHUNK, _RANK), jnp.float32),
            pltpu.VMEM((_CHUNK, _RANK), jnp.float32),
            pltpu.SemaphoreType.DMA,
        ],
        compiler_params=pltpu.CompilerParams(use_tc_tiling_on_sc=False),
    )
    def k(t0_hbm, t1_hbm, idx_hbm, o0_hbm, o1_hbm, idx_v, r0_v, r1_v, sem):
        wid = jax.lax.axis_index("s") * 2 + jax.lax.axis_index("c")
        base_row = wid * (b_per_w // 128)

        @pl.loop(0, n_chunks)
        def _(ci):
            row = base_row + ci * _KROWS
            off = row * 128
            pltpu.sync_copy(idx_hbm.at[pl.ds(row, _KROWS)], idx_v)
            copies = []
            for j in range(_KROWS):
                copies.append(pltpu.async_copy(
                    t0_hbm.at[idx_v.at[j]],
                    r0_v.at[pl.ds(j * 128, 128)], sem))
                copies.append(pltpu.async_copy(
                    t1_hbm.at[idx_v.at[j]],
                    r1_v.at[pl.ds(j * 128, 128)], sem))
            for c in copies:
                c.wait()
            pltpu.sync_copy(r0_v, o0_hbm.at[pl.ds(off, _CHUNK)])
            pltpu.sync_copy(r1_v, o1_hbm.at[pl.ds(off, _CHUNK)])

    idx2d = idx_flat.reshape(m // 128, 128)
    return k(t0, t1, idx2d)


def kernel(nodeIdx, table, W1_0, b1_0, W2_0, b2_0, W1_1, b1_1, W2_1, b2_1):
    r = _RANK
    w1bd = jnp.zeros((2 * r, 2 * r), jnp.float32)
    w1bd = w1bd.at[:r, :r].set(W1_0).at[r:, r:].set(W1_1).astype(jnp.bfloat16)
    w2bd = jnp.zeros((2 * r, 2 * r), jnp.float32)
    w2bd = w2bd.at[:r, :r].set(W2_0).at[r:, r:].set(W2_1).astype(jnp.bfloat16)
    b1cat = jnp.concatenate([b1_0, b1_1]).reshape(1, 2 * r)
    b2cat = jnp.concatenate([b2_0, b2_1]).reshape(1, 2 * r)

    t0, t1 = _transform_table(table, w1bd, b1cat, w2bd, b2cat)

    b, n = nodeIdx.shape
    idx_flat = nodeIdx.reshape(b * n)
    o0, o1 = _sc_gather2(t0, t1, idx_flat)
    return o0.reshape(b, n, r), o1.reshape(b, n, r)


# speedup vs baseline: 2.0928x; 2.0928x over previous
import functools

import jax
import jax.numpy as jnp
from jax.experimental import pallas as pl
from jax.experimental.pallas import tpu as pltpu
from jax.experimental.pallas import tpu_sc as plsc

_LEAF_START = 131071
_RANK = 32
_N8 = 262144
_TBLK = 2048
_NW = 32
_KROWS = 4
_CHUNK = _KROWS * 128


def _transform_body(t_ref, w1_ref, b1_ref, w2_ref, b2_ref, o_ref):
    i = pl.program_id(0)
    x = t_ref[...]
    xb = x.astype(jnp.bfloat16)
    h = jax.lax.dot_general(w1_ref[...], xb, (((1,), (0,)), ((), ())),
                            preferred_element_type=jnp.float32)
    h = jnp.maximum(h + b1_ref[...], 0.0).astype(jnp.bfloat16)
    h2 = jax.lax.dot_general(w2_ref[...], h, (((1,), (0,)), ((), ())),
                             preferred_element_type=jnp.float32)
    h2 = h2 + b2_ref[...]
    cols = i * _TBLK + jax.lax.broadcasted_iota(jnp.int32, (1, _TBLK), 1)
    out_t = jnp.where(cols < _LEAF_START, h2, x)
    out = out_t.T
    o_ref[...] = jnp.concatenate(
        [out, jnp.zeros((_TBLK, 2 * _RANK), jnp.float32)], axis=1)


def _transform_table(table_t, w1t, b1col, w2t, b2col):
    return pl.pallas_call(
        _transform_body,
        grid=(_N8 // _TBLK,),
        in_specs=[
            pl.BlockSpec((2 * _RANK, _TBLK), lambda i: (0, i)),
            pl.BlockSpec((2 * _RANK, 2 * _RANK), lambda i: (0, 0)),
            pl.BlockSpec((2 * _RANK, 1), lambda i: (0, 0)),
            pl.BlockSpec((2 * _RANK, 2 * _RANK), lambda i: (0, 0)),
            pl.BlockSpec((2 * _RANK, 1), lambda i: (0, 0)),
        ],
        out_specs=pl.BlockSpec((_TBLK, 128), lambda i: (i, 0)),
        out_shape=jax.ShapeDtypeStruct((_N8, 128), jnp.float32),
    )(table_t, w1t, b1col, w2t, b2col)


def _sc_gather(tp, idx3d):
    m = idx3d.shape[0] * _CHUNK
    n_chunks = m // _CHUNK
    per_w = n_chunks // _NW
    mesh = plsc.VectorSubcoreMesh(core_axis_name="c", subcore_axis_name="s")

    @functools.partial(
        pl.kernel,
        mesh=mesh,
        out_type=jax.ShapeDtypeStruct((m, 128), jnp.float32),
        scratch_types=[
            pltpu.VMEM((_KROWS, 128), jnp.int32),
            pltpu.VMEM((_CHUNK, 128), jnp.float32),
            pltpu.SemaphoreType.DMA,
        ],
        compiler_params=pltpu.CompilerParams(use_tc_tiling_on_sc=True),
    )
    def k(tp_hbm, idx_hbm, g_hbm, idx_v, r_v, sem):
        wid = jax.lax.axis_index("s") * 2 + jax.lax.axis_index("c")

        @pl.loop(0, per_w)
        def _(ci):
            c = wid * per_w + ci
            pltpu.sync_copy(idx_hbm.at[c], idx_v)
            copies = []
            for j in range(_KROWS):
                copies.append(pltpu.async_copy(
                    tp_hbm.at[idx_v.at[j]],
                    r_v.at[pl.ds(j * 128, 128)], sem))
            for cp in copies:
                cp.wait()
            pltpu.sync_copy(r_v, g_hbm.at[pl.ds(c * _CHUNK, _CHUNK)])

    return k(tp, idx3d)


_BBLK = 16


def _extract_body(g_ref, o0_ref, o1_ref):
    x = g_ref[...]
    for q in range(_BBLK):
        xt = x[q * 512:(q + 1) * 512, :2 * _RANK].T
        o0_ref[q] = xt[:_RANK]
        o1_ref[q] = xt[_RANK:]


def _extract(g, b, n):
    return pl.pallas_call(
        _extract_body,
        grid=(b // _BBLK,),
        in_specs=[pl.BlockSpec((_BBLK * n, 128), lambda i: (i, 0))],
        out_specs=[
            pl.BlockSpec((_BBLK, _RANK, n), lambda i: (i, 0, 0)),
            pl.BlockSpec((_BBLK, _RANK, n), lambda i: (i, 0, 0)),
        ],
        out_shape=[
            jax.ShapeDtypeStruct((b, _RANK, n), jnp.float32),
            jax.ShapeDtypeStruct((b, _RANK, n), jnp.float32),
        ],
    )(g)


def kernel(nodeIdx, table, W1_0, b1_0, W2_0, b2_0, W1_1, b1_1, W2_1, b2_1):
    r = _RANK
    w1bd = jnp.zeros((2 * r, 2 * r), jnp.float32)
    w1bd = w1bd.at[:r, :r].set(W1_0).at[r:, r:].set(W1_1)
    w2bd = jnp.zeros((2 * r, 2 * r), jnp.float32)
    w2bd = w2bd.at[:r, :r].set(W2_0).at[r:, r:].set(W2_1)
    w1t = w1bd.T.astype(jnp.bfloat16)
    w2t = w2bd.T.astype(jnp.bfloat16)
    b1col = jnp.concatenate([b1_0, b1_1]).reshape(2 * r, 1)
    b2col = jnp.concatenate([b2_0, b2_1]).reshape(2 * r, 1)

    tp = _transform_table(table.T, w1t, b1col, w2t, b2col)

    b, n = nodeIdx.shape
    idx3d = nodeIdx.reshape(b * n // _CHUNK, _KROWS, 128)
    g = _sc_gather(tp, idx3d)
    o0t, o1t = _extract(g, b, n)
    return (jnp.transpose(o0t, (0, 2, 1)), jnp.transpose(o1t, (0, 2, 1)))


# speedup vs baseline: 2.1425x; 1.0237x over previous
import functools

import jax
import jax.numpy as jnp
from jax.experimental import pallas as pl
from jax.experimental.pallas import tpu as pltpu
from jax.experimental.pallas import tpu_sc as plsc

_LEAF_START = 131071
_RANK = 32
_N8 = 262144
_TBLK = 2048
_NW = 32
_KROWS = 4
_CHUNK = _KROWS * 128


def _transform_body(t_ref, w1_ref, b1_ref, w2_ref, b2_ref, o_ref):
    i = pl.program_id(0)
    x = t_ref[...]
    xb = x.astype(jnp.bfloat16)
    h = jax.lax.dot_general(w1_ref[...], xb, (((1,), (0,)), ((), ())),
                            preferred_element_type=jnp.float32)
    h = jnp.maximum(h + b1_ref[...], 0.0).astype(jnp.bfloat16)
    h2 = jax.lax.dot_general(w2_ref[...], h, (((1,), (0,)), ((), ())),
                             preferred_element_type=jnp.float32)
    h2 = h2 + b2_ref[...]
    cols = i * _TBLK + jax.lax.broadcasted_iota(jnp.int32, (1, _TBLK), 1)
    out_t = jnp.where(cols < _LEAF_START, h2, x)
    out = out_t.T
    o_ref[...] = jnp.concatenate(
        [out, jnp.zeros((_TBLK, 2 * _RANK), jnp.float32)], axis=1)


def _transform_table(table_t, w1t, b1col, w2t, b2col):
    return pl.pallas_call(
        _transform_body,
        grid=(_N8 // _TBLK,),
        in_specs=[
            pl.BlockSpec((2 * _RANK, _TBLK), lambda i: (0, i)),
            pl.BlockSpec((2 * _RANK, 2 * _RANK), lambda i: (0, 0)),
            pl.BlockSpec((2 * _RANK, 1), lambda i: (0, 0)),
            pl.BlockSpec((2 * _RANK, 2 * _RANK), lambda i: (0, 0)),
            pl.BlockSpec((2 * _RANK, 1), lambda i: (0, 0)),
        ],
        out_specs=pl.BlockSpec((_TBLK, 128), lambda i: (i, 0)),
        out_shape=jax.ShapeDtypeStruct((_N8, 128), jnp.float32),
    )(table_t, w1t, b1col, w2t, b2col)


def _sc_gather(tp, idx3d):
    m = idx3d.shape[0] * _CHUNK
    n_chunks = m // _CHUNK
    per_w = n_chunks // _NW
    mesh = plsc.VectorSubcoreMesh(core_axis_name="c", subcore_axis_name="s")

    @functools.partial(
        pl.kernel,
        mesh=mesh,
        out_type=jax.ShapeDtypeStruct((m, 128), jnp.float32),
        scratch_types=[
            pltpu.VMEM((_KROWS, 128), jnp.int32),
            pltpu.VMEM((_CHUNK, 128), jnp.float32),
            pltpu.SemaphoreType.DMA,
        ],
        compiler_params=pltpu.CompilerParams(use_tc_tiling_on_sc=True),
    )
    def k(tp_hbm, idx_hbm, g_hbm, idx_v, r_v, sem):
        wid = jax.lax.axis_index("s") * 2 + jax.lax.axis_index("c")

        @pl.loop(0, per_w)
        def _(ci):
            c = wid * per_w + ci
            pltpu.sync_copy(idx_hbm.at[c], idx_v)
            copies = []
            for j in range(_KROWS):
                copies.append(pltpu.async_copy(
                    tp_hbm.at[idx_v.at[j]],
                    r_v.at[pl.ds(j * 128, 128)], sem))
            for cp in copies:
                cp.wait()
            pltpu.sync_copy(r_v, g_hbm.at[pl.ds(c * _CHUNK, _CHUNK)])

    return k(tp, idx3d)


_BBLK = 16
_NSLICE = 4


def _extract_body(g_ref, o0_ref, o1_ref):
    x = g_ref[...]
    for q in range(_BBLK):
        xt = x[q * 512:(q + 1) * 512, :2 * _RANK].T
        o0_ref[q] = xt[:_RANK]
        o1_ref[q] = xt[_RANK:]


def _extract_first_body(g_ref, o0_ref, o1_ref):
    _extract_body(g_ref, o0_ref, o1_ref)


def _extract_slice_body(g_ref, p0_ref, p1_ref, o0_ref, o1_ref):
    del p0_ref, p1_ref
    _extract_body(g_ref, o0_ref, o1_ref)


def _extract_first(g0, b, n, bs):
    return pl.pallas_call(
        _extract_first_body,
        grid=(bs // _BBLK,),
        in_specs=[pl.BlockSpec((_BBLK * n, 128), lambda i: (i, 0))],
        out_specs=[
            pl.BlockSpec((_BBLK, _RANK, n), lambda i: (i, 0, 0)),
            pl.BlockSpec((_BBLK, _RANK, n), lambda i: (i, 0, 0)),
        ],
        out_shape=[
            jax.ShapeDtypeStruct((b, _RANK, n), jnp.float32),
            jax.ShapeDtypeStruct((b, _RANK, n), jnp.float32),
        ],
    )(g0)


def _extract_slice(g_s, p0, p1, s, n, bs):
    off = s * (bs // _BBLK)
    return pl.pallas_call(
        _extract_slice_body,
        grid=(bs // _BBLK,),
        in_specs=[
            pl.BlockSpec((_BBLK * n, 128), lambda i: (i, 0)),
            pl.BlockSpec((1, 8, 128), lambda i: (0, 0, 0)),
            pl.BlockSpec((1, 8, 128), lambda i: (0, 0, 0)),
        ],
        out_specs=[
            pl.BlockSpec((_BBLK, _RANK, n), lambda i, o=off: (o + i, 0, 0)),
            pl.BlockSpec((_BBLK, _RANK, n), lambda i, o=off: (o + i, 0, 0)),
        ],
        out_shape=[
            jax.ShapeDtypeStruct(p0.shape, jnp.float32),
            jax.ShapeDtypeStruct(p1.shape, jnp.float32),
        ],
        input_output_aliases={1: 0, 2: 1},
    )(g_s, p0, p1)


def kernel(nodeIdx, table, W1_0, b1_0, W2_0, b2_0, W1_1, b1_1, W2_1, b2_1):
    r = _RANK
    w1bd = jnp.zeros((2 * r, 2 * r), jnp.float32)
    w1bd = w1bd.at[:r, :r].set(W1_0).at[r:, r:].set(W1_1)
    w2bd = jnp.zeros((2 * r, 2 * r), jnp.float32)
    w2bd = w2bd.at[:r, :r].set(W2_0).at[r:, r:].set(W2_1)
    w1t = w1bd.T.astype(jnp.bfloat16)
    w2t = w2bd.T.astype(jnp.bfloat16)
    b1col = jnp.concatenate([b1_0, b1_1]).reshape(2 * r, 1)
    b2col = jnp.concatenate([b2_0, b2_1]).reshape(2 * r, 1)

    tp = _transform_table(table.T, w1t, b1col, w2t, b2col)

    b, n = nodeIdx.shape
    idx3d = nodeIdx.reshape(b * n // _CHUNK, _KROWS, 128)
    bs = b // _NSLICE
    cs = idx3d.shape[0] // _NSLICE
    gs = [_sc_gather(tp, idx3d[s * cs:(s + 1) * cs]) for s in range(_NSLICE)]
    o0t, o1t = _extract_first(gs[0], b, n, bs)
    for s in range(1, _NSLICE):
        o0t, o1t = _extract_slice(gs[s], o0t, o1t, s, n, bs)
    return (jnp.transpose(o0t, (0, 2, 1)), jnp.transpose(o1t, (0, 2, 1)))
